# baseline (speedup 1.0000x reference)
import functools

import jax
import jax.numpy as jnp
from jax import lax
from jax.experimental import pallas as pl
from jax.experimental.pallas import tpu as pltpu
from jax.experimental.pallas import tpu_sc as plsc

_TAU = 0.7
_FGT = 0.1
_MASS = 0.2
_CTX = 1.8
_NI, _NC, _H, _W = 4, 20, 512, 512
_CP = 32
_NCORE, _NSUB = 2, 16
_NW = _NCORE * _NSUB
_R = 20000
_RP = 20480
_RPW = _RP // _NW
_GRP = 16
_NGRP = _RPW // _GRP
_ZROW = _NI * _H * _W


def _ii_body(x_ref, o_ref):
    x = x_ref[0, 0]
    mx = jnp.max(x)
    fg = jnp.where(x >= _FGT * mx, x, 0.0)
    r = lax.broadcasted_iota(jnp.int32, (_H, _H), 0)
    c = lax.broadcasted_iota(jnp.int32, (_H, _H), 1)
    d = r - c
    lo = jnp.clip(d + 1, 0, 1).astype(jnp.float32)
    up = jnp.clip(1 - d, 0, 1).astype(jnp.float32)
    t = jnp.dot(lo, fg, preferred_element_type=jnp.float32)
    o_ref[0, 0] = jnp.dot(t, up, preferred_element_type=jnp.float32)


_ii_call = pl.pallas_call(
    _ii_body,
    out_shape=jax.ShapeDtypeStruct((_NI, _NC, _H, _W), jnp.float32),
    grid=(_NI, _NC),
    in_specs=[pl.BlockSpec((1, 1, _H, _W), lambda n, c: (n, c, 0, 0))],
    out_specs=pl.BlockSpec((1, 1, _H, _W), lambda n, c: (n, c, 0, 0)),
)

_YB = 8
_NBLK = _NI * (_H // _YB)
_TROWS = (_NBLK + 1) * _YB * _W


def _tr_body(x_ref, o_ref):
    g = pl.program_id(0)

    @pl.when(g >= _NBLK)
    def _():
        o_ref[...] = jnp.zeros((_YB * _W, _CP), jnp.float32)

    @pl.when(g < _NBLK)
    def _():
        r = lax.broadcasted_iota(jnp.int32, (_CP, _CP), 0)
        c = lax.broadcasted_iota(jnp.int32, (_CP, _CP), 1)
        eye = jnp.clip(1 - jnp.abs(r - c), 0, 1).astype(jnp.float32)
        zpad = jnp.zeros((_CP - _NC, _W), jnp.float32)
        for yy in range(_YB):
            m = jnp.concatenate([x_ref[0, :, yy, :], zpad], axis=0)
            o_ref[yy * _W:(yy + 1) * _W, :] = lax.dot_general(
                m, eye, (((0,), (0,)), ((), ())),
                precision=lax.Precision.HIGHEST,
                preferred_element_type=jnp.float32)


_tr_call = pl.pallas_call(
    _tr_body,
    out_shape=jax.ShapeDtypeStruct((_TROWS, _CP), jnp.float32),
    grid=(_NBLK + 1,),
    in_specs=[pl.BlockSpec(
        (1, _NC, _YB, _W),
        lambda g: (jnp.minimum(g // (_H // _YB), _NI - 1), 0,
                   g % (_H // _YB), 0))],
    out_specs=pl.BlockSpec((_YB * _W, _CP), lambda g: (g, 0)),
)


def _bc(v16, j):
    idx = jnp.full((16, 1), j, dtype=jnp.int32)
    return lax.gather(
        v16, idx,
        lax.GatherDimensionNumbers(offset_dims=(), collapsed_slice_dims=(0,),
                                   start_index_map=(0,)),
        (1,), mode=lax.GatherScatterMode.PROMISE_IN_BOUNDS)


_mesh = plsc.VectorSubcoreMesh(core_axis_name="c", subcore_axis_name="s")


@functools.partial(
    pl.kernel,
    out_type=(jax.ShapeDtypeStruct((_NW, _RPW * _CP), jnp.float32),
              jax.ShapeDtypeStruct((_NW, 128), jnp.float32)),
    mesh=_mesh,
    compiler_params=pltpu.CompilerParams(use_tc_tiling_on_sc=False),
    scratch_types=[
        pltpu.VMEM((_RPW * 8,), jnp.int32),
        pltpu.VMEM((_RPW * 3,), jnp.float32),
        pltpu.VMEM((128,), jnp.float32),
        pltpu.VMEM((_GRP * 8, _CP), jnp.float32),
        pltpu.VMEM((_RPW * _CP,), jnp.float32),
        pltpu.VMEM((128,), jnp.float32),
        pltpu.SemaphoreType.DMA,
    ],
)
def _csc_pass1(tbl, idxh, rph, ith, score_h, part_h,
               idx_v, rp_v, it_v, rows_v, sc_v, pt_v, sem):
    wid = lax.axis_index("s") * _NCORE + lax.axis_index("c")
    pltpu.sync_copy(idxh.at[wid], idx_v)
    pltpu.sync_copy(rph.at[wid], rp_v)
    pltpu.sync_copy(ith, it_v)
    it_rows = [(it_v[n * 32:n * 32 + 16], it_v[n * 32 + 16:n * 32 + 32])
               for n in range(4)]
    zero = jnp.zeros((16,), jnp.float32)

    def grp(g, acc):
        pltpu.async_copy(tbl.at[idx_v.at[pl.ds(g * (_GRP * 8), _GRP * 8)]],
                         rows_v, sem).wait()
        r16 = rp_v[pl.ds(g * _GRP, _GRP)]
        f16 = rp_v[pl.ds(_RPW + g * _GRP, _GRP)]
        b16 = rp_v[pl.ds(2 * _RPW + g * _GRP, _GRP)]
        base = g * (_GRP * _CP)
        acc = list(acc)
        for j in range(_GRP):
            rb = j * 8
            rin = _bc(r16, j)
            rfr = _bc(f16, j)
            bb = _bc(b16, j)
            for h in range(2):
                s0 = h * 16
                va = rows_v[rb + 0, s0:s0 + 16]
                vb = rows_v[rb + 1, s0:s0 + 16]
                vc = rows_v[rb + 2, s0:s0 + 16]
                vd = rows_v[rb + 3, s0:s0 + 16]
                ve = rows_v[rb + 4, s0:s0 + 16]
                vf = rows_v[rb + 5, s0:s0 + 16]
                vg = rows_v[rb + 6, s0:s0 + 16]
                vh = rows_v[rb + 7, s0:s0 + 16]
                m_in = va - vb - vc + vd
                m_ctx = ve - vf - vg + vh
                m_fr = jnp.maximum(m_ctx - m_in, 0.0)
                s = m_in * rin - m_fr * rfr
                oh = [jnp.maximum(1.0 - jnp.abs(bb - float(n)), 0.0)
                      for n in range(4)]
                itr = oh[0] * it_rows[0][h]
                for n in range(1, 4):
                    itr = itr + oh[n] * it_rows[n][h]
                gate = jnp.logical_and(m_in * itr >= _MASS, m_in >= 0.0)
                sp = jnp.where(gate, s, jnp.minimum(s, 0.0))
                sc_v[pl.ds(base + j * _CP + s0, 16)] = sp
                ab = jnp.abs(s)
                for n in range(4):
                    k = n * 2 + h
                    acc[k] = jnp.maximum(acc[k], ab * oh[n])
        return tuple(acc)

    acc = lax.fori_loop(0, _NGRP, grp, tuple(zero for _ in range(8)))
    for n in range(4):
        pt_v[n * 32:n * 32 + 16] = acc[n * 2]
        pt_v[n * 32 + 16:n * 32 + 32] = acc[n * 2 + 1]
    pltpu.sync_copy(sc_v, score_h.at[wid])
    pltpu.sync_copy(pt_v, part_h.at[wid])


@functools.partial(
    pl.kernel,
    out_type=jax.ShapeDtypeStruct((_NW, _RPW * _CP), jnp.float32),
    mesh=_mesh,
    compiler_params=pltpu.CompilerParams(use_tc_tiling_on_sc=False),
    scratch_types=[
        pltpu.VMEM((_RPW * _CP,), jnp.float32),
        pltpu.VMEM((_NW, 128), jnp.float32),
        pltpu.VMEM((_RPW * 3,), jnp.float32),
        pltpu.VMEM((128,), jnp.float32),
        pltpu.VMEM((_RPW * _CP,), jnp.float32),
    ],
)
def _csc_pass2(score_h, part_h, rph, labh, w_h, sc_v, pt_v, rp_v, lab_v, w_v):
    wid = lax.axis_index("s") * _NCORE + lax.axis_index("c")
    pltpu.sync_copy(score_h.at[wid], sc_v)
    pltpu.sync_copy(part_h, pt_v)
    pltpu.sync_copy(rph.at[wid], rp_v)
    pltpu.sync_copy(labh, lab_v)
    iam = []
    for n in range(4):
        for h in range(2):
            o = n * 32 + h * 16
            m = pt_v[0, o:o + 16]
            for t in range(1, _NW):
                m = jnp.maximum(m, pt_v[t, o:o + 16])
            iam.append(1.0 / jnp.maximum(m, 1e-6))
    lab_rows = [(lab_v[n * 32:n * 32 + 16], lab_v[n * 32 + 16:n * 32 + 32])
                for n in range(4)]

    def grp(g, carry):
        b16 = rp_v[pl.ds(2 * _RPW + g * _GRP, _GRP)]
        base = g * (_GRP * _CP)
        for j in range(_GRP):
            bb = _bc(b16, j)
            for h in range(2):
                off = base + j * _CP + h * 16
                s = sc_v[pl.ds(off, 16)]
                oh = [jnp.maximum(1.0 - jnp.abs(bb - float(n)), 0.0)
                      for n in range(4)]
                ia = oh[0] * iam[h]
                lb = oh[0] * lab_rows[0][h]
                for n in range(1, 4):
                    ia = ia + oh[n] * iam[n * 2 + h]
                    lb = lb + oh[n] * lab_rows[n][h]
                w = jnp.clip(s * ia, -1.0, 1.0)
                w = jnp.where(w >= _TAU, 1.0, w)
                w = jnp.where(lb > 0.5, w, 1.0)
                w_v[pl.ds(off, 16)] = w
        return carry

    lax.fori_loop(0, _NGRP, grp, 0)
    pltpu.sync_copy(w_v, w_h.at[wid])


def kernel(cpgs, labels, preds, rois):
    ii = _ii_call(cpgs)
    total = ii[:, :, -1, -1]
    tbl = _tr_call(ii)

    b = rois[:, 0].astype(jnp.int32)
    x1 = jnp.clip(jnp.floor(rois[:, 1]), 0, _W - 1).astype(jnp.int32)
    y1 = jnp.clip(jnp.floor(rois[:, 2]), 0, _H - 1).astype(jnp.int32)
    x2 = jnp.clip(jnp.ceil(rois[:, 3]), 0, _W - 1).astype(jnp.int32)
    y2 = jnp.clip(jnp.ceil(rois[:, 4]), 0, _H - 1).astype(jnp.int32)
    x2 = jnp.maximum(x2, x1)
    y2 = jnp.maximum(y2, y1)
    cx = (x1 + x2).astype(jnp.float32) * 0.5
    cy = (y1 + y2).astype(jnp.float32) * 0.5
    hw = (x2 - x1 + 1).astype(jnp.float32) * 0.5 * _CTX
    hh = (y2 - y1 + 1).astype(jnp.float32) * 0.5 * _CTX
    cx1 = jnp.clip(jnp.floor(cx - hw), 0, _W - 1).astype(jnp.int32)
    cy1 = jnp.clip(jnp.floor(cy - hh), 0, _H - 1).astype(jnp.int32)
    cx2 = jnp.clip(jnp.ceil(cx + hw), 0, _W - 1).astype(jnp.int32)
    cy2 = jnp.clip(jnp.ceil(cy + hh), 0, _H - 1).astype(jnp.int32)

    def rowidx(py, px):
        valid = jnp.logical_and(py > 0, px > 0)
        return jnp.where(valid, (b * _H + (py - 1)) * _W + (px - 1), _ZROW)

    idx8 = jnp.stack([
        rowidx(y2 + 1, x2 + 1), rowidx(y1, x2 + 1),
        rowidx(y2 + 1, x1), rowidx(y1, x1),
        rowidx(cy2 + 1, cx2 + 1), rowidx(cy1, cx2 + 1),
        rowidx(cy2 + 1, cx1), rowidx(cy1, cx1),
    ], axis=1)
    idx8 = jnp.pad(idx8, ((0, _RP - _R), (0, 0)), constant_values=_ZROW)

    a_in = ((x2 - x1 + 1) * (y2 - y1 + 1)).astype(jnp.float32)
    a_ctx = ((cx2 - cx1 + 1) * (cy2 - cy1 + 1)).astype(jnp.float32)
    a_fr = jnp.maximum(a_ctx - a_in, 1.0)
    r_in = 1.0 / jnp.sqrt(a_in)
    r_fr = 1.0 / jnp.sqrt(a_fr)
    pad = lambda v: jnp.pad(v, (0, _RP - _R))
    rp = jnp.stack([pad(r_in).reshape(_NW, _RPW),
                    pad(r_fr).reshape(_NW, _RPW),
                    pad(b.astype(jnp.float32)).reshape(_NW, _RPW)],
                   axis=1).reshape(_NW, 3 * _RPW)

    inv_total = 1.0 / jnp.maximum(total, 1e-6)
    itf = jnp.pad(inv_total, ((0, 0), (0, _CP - _NC))).reshape(128)
    labf = jnp.pad(labels, ((0, 0), (0, _CP - _NC))).reshape(128)

    score_flat, part = _csc_pass1(tbl, idx8.reshape(_NW, _RPW * 8), rp, itf)
    w_flat = _csc_pass2(score_flat, part, rp, labf)
    W_out = w_flat.reshape(_RP, _CP)[:_R, :_NC]
    return (W_out, labels, jnp.zeros_like(labels))


# speedup vs baseline: 1.4486x; 1.4486x over previous
import functools

import jax
import jax.numpy as jnp
from jax import lax
from jax.experimental import pallas as pl
from jax.experimental.pallas import tpu as pltpu
from jax.experimental.pallas import tpu_sc as plsc

_TAU = 0.7
_FGT = 0.1
_MASS = 0.2
_CTX = 1.8
_NI, _NC, _H, _W = 4, 20, 512, 512
_CP = 32
_NCORE, _NSUB = 2, 16
_NW = _NCORE * _NSUB
_R = 20000
_RP = 20480
_RPW = _RP // _NW
_GRP = 16
_NGRP = _RPW // _GRP


def _ii_body(x_ref, o_ref):
    x = x_ref[0, 0]
    mx = jnp.max(x)
    fg = jnp.where(x >= _FGT * mx, x, 0.0)
    r = lax.broadcasted_iota(jnp.int32, (_H, _H), 0)
    c = lax.broadcasted_iota(jnp.int32, (_H, _H), 1)
    d = r - c
    lo = jnp.clip(d + 1, 0, 1).astype(jnp.float32)
    up = jnp.clip(1 - d, 0, 1).astype(jnp.float32)
    t = jnp.dot(lo, fg, preferred_element_type=jnp.float32)
    o_ref[0, 0] = jnp.dot(t, up, preferred_element_type=jnp.float32)


_ii_call = pl.pallas_call(
    _ii_body,
    out_shape=jax.ShapeDtypeStruct((_NI, _NC, _H, _W), jnp.float32),
    grid=(_NI, _NC),
    in_specs=[pl.BlockSpec((1, 1, _H, _W), lambda n, c: (n, c, 0, 0))],
    out_specs=pl.BlockSpec((1, 1, _H, _W), lambda n, c: (n, c, 0, 0)),
)

_YQ = 4
_QB = 16
_NBLK = _NI * (_H // (_YQ * _QB))
_TBROWS = _QB * _W
_TROWS = (_NBLK + 1) * _TBROWS
_ZROW = _NI * (_H // _YQ) * _W


def _tr_body(x_ref, o_ref):
    g = pl.program_id(0)

    @pl.when(g >= _NBLK)
    def _():
        o_ref[...] = jnp.zeros((_TBROWS, 4 * _CP), jnp.float32)

    @pl.when(g < _NBLK)
    def _():
        ri = lax.broadcasted_iota(jnp.int32, (_YQ * _NC, 4 * _CP), 0)
        ci = lax.broadcasted_iota(jnp.int32, (_YQ * _NC, 4 * _CP), 1)
        tgt = (ri % _YQ) * _CP + ri // _YQ
        sel = jnp.clip(1 - jnp.abs(tgt - ci), 0, 1).astype(jnp.float32)
        for yq in range(_QB):
            m = x_ref[0, :, yq * _YQ:(yq + 1) * _YQ, :].reshape(
                _NC * _YQ, _W)
            o_ref[yq * _W:(yq + 1) * _W, :] = lax.dot_general(
                m, sel, (((0,), (0,)), ((), ())),
                precision=lax.Precision.HIGHEST,
                preferred_element_type=jnp.float32)


_tr_call = pl.pallas_call(
    _tr_body,
    out_shape=jax.ShapeDtypeStruct((_TROWS, 4 * _CP), jnp.float32),
    grid=(_NBLK + 1,),
    in_specs=[pl.BlockSpec(
        (1, _NC, _YQ * _QB, _W),
        lambda g: (jnp.minimum(g // (_H // (_YQ * _QB)), _NI - 1), 0,
                   g % (_H // (_YQ * _QB)), 0))],
    out_specs=pl.BlockSpec((_TBROWS, 4 * _CP), lambda g: (g, 0)),
)


def _bc(v16, j):
    idx = jnp.full((16, 1), j, dtype=jnp.int32)
    return lax.gather(
        v16, idx,
        lax.GatherDimensionNumbers(offset_dims=(), collapsed_slice_dims=(0,),
                                   start_index_map=(0,)),
        (1,), mode=lax.GatherScatterMode.PROMISE_IN_BOUNDS)


_mesh = plsc.VectorSubcoreMesh(core_axis_name="c", subcore_axis_name="s")


@functools.partial(
    pl.kernel,
    out_type=(jax.ShapeDtypeStruct((_NW, _RPW * _CP), jnp.float32),
              jax.ShapeDtypeStruct((_NW, 128), jnp.float32)),
    mesh=_mesh,
    scratch_types=[
        pltpu.VMEM((_RPW * 8,), jnp.int32),
        pltpu.VMEM((_RPW * 19,), jnp.float32),
        pltpu.VMEM((128,), jnp.float32),
        pltpu.VMEM((_GRP * 8, 4 * _CP), jnp.float32),
        pltpu.VMEM((_RPW * _CP,), jnp.float32),
        pltpu.VMEM((128,), jnp.float32),
        pltpu.SemaphoreType.DMA,
    ],
)
def _csc_pass1(tbl, idxh, rph, ith, score_h, part_h,
               idx_v, rp_v, it_v, rows_v, sc_v, pt_v, sem):
    wid = lax.axis_index("s") * _NCORE + lax.axis_index("c")
    pltpu.sync_copy(idxh.at[wid], idx_v)
    pltpu.sync_copy(rph.at[wid], rp_v)
    pltpu.sync_copy(ith, it_v)
    it_rows = [(it_v[n * 32:n * 32 + 16], it_v[n * 32 + 16:n * 32 + 32])
               for n in range(4)]
    zero = jnp.zeros((16,), jnp.float32)
    iota = lax.iota(jnp.int32, 16)

    def grp(g, acc):
        pltpu.async_copy(tbl.at[idx_v.at[pl.ds(g * (_GRP * 8), _GRP * 8)]],
                         rows_v, sem).wait()
        r16 = rp_v[pl.ds(g * _GRP, _GRP)]
        f16 = rp_v[pl.ds(_RPW + g * _GRP, _GRP)]
        b16 = rp_v[pl.ds(2 * _RPW + g * _GRP, _GRP)]
        q16a = [rp_v[pl.ds((3 + k) * _RPW + g * _GRP, _GRP)] for k in range(8)]
        q16b = [rp_v[pl.ds((11 + k) * _RPW + g * _GRP, _GRP)]
                for k in range(8)]
        base = g * (_GRP * _CP)
        acc = list(acc)
        for j in range(_GRP):
            rb = j * 8
            rin = _bc(r16, j)
            rfr = _bc(f16, j)
            bb = _bc(b16, j)
            fa = [_bc(q16a[k], j) for k in range(8)]
            fb = [_bc(q16b[k], j) for k in range(8)]
            for h in range(2):
                s0 = h * 16

                def pick(k, _s0=s0, _rb=rb, _fa=fa, _fb=fb):
                    r = rb + k
                    v0 = rows_v[r, _s0:_s0 + 16]
                    v1 = rows_v[r, 32 + _s0:32 + _s0 + 16]
                    v2 = rows_v[r, 64 + _s0:64 + _s0 + 16]
                    v3 = rows_v[r, 96 + _s0:96 + _s0 + 16]
                    lo = v0 + _fa[k] * (v1 - v0)
                    hi = v2 + _fa[k] * (v3 - v2)
                    return lo + _fb[k] * (hi - lo)

                cv = [pick(k) for k in range(8)]
                va, vb, vc, vd, ve, vf, vg, vh = cv
                m_in = va - vb - vc + vd
                m_ctx = ve - vf - vg + vh
                m_fr = jnp.maximum(m_ctx - m_in, 0.0)
                s = m_in * rin - m_fr * rfr
                oh = [jnp.maximum(1.0 - jnp.abs(bb - float(n)), 0.0)
                      for n in range(4)]
                itr = oh[0] * it_rows[0][h]
                for n in range(1, 4):
                    itr = itr + oh[n] * it_rows[n][h]
                gate = jnp.logical_and(m_in * itr >= _MASS, m_in >= 0.0)
                sp = jnp.where(gate, s, jnp.minimum(s, 0.0))
                sc_v[pl.ds(base + j * _CP + s0, 16)] = sp
                ab = jnp.abs(s)
                for n in range(4):
                    k = n * 2 + h
                    acc[k] = jnp.maximum(acc[k], ab * oh[n])
        return tuple(acc)

    acc = lax.fori_loop(0, _NGRP, grp, tuple(zero for _ in range(8)))
    for n in range(4):
        pt_v[n * 32:n * 32 + 16] = acc[n * 2]
        pt_v[n * 32 + 16:n * 32 + 32] = acc[n * 2 + 1]
    pltpu.sync_copy(sc_v, score_h.at[wid])
    pltpu.sync_copy(pt_v, part_h.at[wid])


@functools.partial(
    pl.kernel,
    out_type=jax.ShapeDtypeStruct((_NW, _RPW * _CP), jnp.float32),
    mesh=_mesh,
    scratch_types=[
        pltpu.VMEM((_RPW * _CP,), jnp.float32),
        pltpu.VMEM((_NW, 128), jnp.float32),
        pltpu.VMEM((_RPW * 19,), jnp.float32),
        pltpu.VMEM((128,), jnp.float32),
        pltpu.VMEM((_RPW * _CP,), jnp.float32),
    ],
)
def _csc_pass2(score_h, part_h, rph, labh, w_h, sc_v, pt_v, rp_v, lab_v, w_v):
    wid = lax.axis_index("s") * _NCORE + lax.axis_index("c")
    pltpu.sync_copy(score_h.at[wid], sc_v)
    pltpu.sync_copy(part_h, pt_v)
    pltpu.sync_copy(rph.at[wid], rp_v)
    pltpu.sync_copy(labh, lab_v)
    iam = []
    for n in range(4):
        for h in range(2):
            o = n * 32 + h * 16
            m = pt_v[0, o:o + 16]
            for t in range(1, _NW):
                m = jnp.maximum(m, pt_v[t, o:o + 16])
            iam.append(1.0 / jnp.maximum(m, 1e-6))
    lab_rows = [(lab_v[n * 32:n * 32 + 16], lab_v[n * 32 + 16:n * 32 + 32])
                for n in range(4)]

    def grp(g, carry):
        b16 = rp_v[pl.ds(2 * _RPW + g * _GRP, _GRP)]
        base = g * (_GRP * _CP)
        for j in range(_GRP):
            bb = _bc(b16, j)
            for h in range(2):
                off = base + j * _CP + h * 16
                s = sc_v[pl.ds(off, 16)]
                oh = [jnp.maximum(1.0 - jnp.abs(bb - float(n)), 0.0)
                      for n in range(4)]
                ia = oh[0] * iam[h]
                lb = oh[0] * lab_rows[0][h]
                for n in range(1, 4):
                    ia = ia + oh[n] * iam[n * 2 + h]
                    lb = lb + oh[n] * lab_rows[n][h]
                w = jnp.clip(s * ia, -1.0, 1.0)
                w = jnp.where(w >= _TAU, 1.0, w)
                w = jnp.where(lb > 0.5, w, 1.0)
                w_v[pl.ds(off, 16)] = w
        return carry

    lax.fori_loop(0, _NGRP, grp, 0)
    pltpu.sync_copy(w_v, w_h.at[wid])


def kernel(cpgs, labels, preds, rois):
    ii = _ii_call(cpgs)
    total = ii[:, :, -1, -1]
    tbl = _tr_call(ii)

    b = rois[:, 0].astype(jnp.int32)
    x1 = jnp.clip(jnp.floor(rois[:, 1]), 0, _W - 1).astype(jnp.int32)
    y1 = jnp.clip(jnp.floor(rois[:, 2]), 0, _H - 1).astype(jnp.int32)
    x2 = jnp.clip(jnp.ceil(rois[:, 3]), 0, _W - 1).astype(jnp.int32)
    y2 = jnp.clip(jnp.ceil(rois[:, 4]), 0, _H - 1).astype(jnp.int32)
    x2 = jnp.maximum(x2, x1)
    y2 = jnp.maximum(y2, y1)
    cx = (x1 + x2).astype(jnp.float32) * 0.5
    cy = (y1 + y2).astype(jnp.float32) * 0.5
    hw = (x2 - x1 + 1).astype(jnp.float32) * 0.5 * _CTX
    hh = (y2 - y1 + 1).astype(jnp.float32) * 0.5 * _CTX
    cx1 = jnp.clip(jnp.floor(cx - hw), 0, _W - 1).astype(jnp.int32)
    cy1 = jnp.clip(jnp.floor(cy - hh), 0, _H - 1).astype(jnp.int32)
    cx2 = jnp.clip(jnp.ceil(cx + hw), 0, _W - 1).astype(jnp.int32)
    cy2 = jnp.clip(jnp.ceil(cy + hh), 0, _H - 1).astype(jnp.int32)

    def rowidx(py, px):
        valid = jnp.logical_and(py > 0, px > 0)
        row = (b * (_H // _YQ) + (py - 1) // _YQ) * _W + (px - 1)
        q = jnp.where(valid, (py - 1) % _YQ, 0)
        return (jnp.where(valid, row, _ZROW),
                (q % 2).astype(jnp.float32), (q // 2).astype(jnp.float32))

    corners = [
        rowidx(y2 + 1, x2 + 1), rowidx(y1, x2 + 1),
        rowidx(y2 + 1, x1), rowidx(y1, x1),
        rowidx(cy2 + 1, cx2 + 1), rowidx(cy1, cx2 + 1),
        rowidx(cy2 + 1, cx1), rowidx(cy1, cx1),
    ]
    idx8 = jnp.stack([cr[0] for cr in corners], axis=1)
    idx8 = jnp.pad(idx8, ((0, _RP - _R), (0, 0)), constant_values=_ZROW)

    a_in = ((x2 - x1 + 1) * (y2 - y1 + 1)).astype(jnp.float32)
    a_ctx = ((cx2 - cx1 + 1) * (cy2 - cy1 + 1)).astype(jnp.float32)
    a_fr = jnp.maximum(a_ctx - a_in, 1.0)
    r_in = 1.0 / jnp.sqrt(a_in)
    r_fr = 1.0 / jnp.sqrt(a_fr)
    pad = lambda v: jnp.pad(v, (0, _RP - _R))
    slots = ([r_in, r_fr, b.astype(jnp.float32)]
             + [cr[1] for cr in corners] + [cr[2] for cr in corners])
    rp = jnp.stack([pad(v).reshape(_NW, _RPW) for v in slots],
                   axis=1).reshape(_NW, 19 * _RPW)

    inv_total = 1.0 / jnp.maximum(total, 1e-6)
    itf = jnp.pad(inv_total, ((0, 0), (0, _CP - _NC))).reshape(128)
    labf = jnp.pad(labels, ((0, 0), (0, _CP - _NC))).reshape(128)

    score_flat, part = _csc_pass1(tbl, idx8.reshape(_NW, _RPW * 8), rp, itf)
    w_flat = _csc_pass2(score_flat, part, rp, labf)
    W_out = w_flat.reshape(_RP, _CP)[:_R, :_NC]
    return (W_out, labels, jnp.zeros_like(labels))


# speedup vs baseline: 1.6045x; 1.1076x over previous
import functools

import jax
import jax.numpy as jnp
from jax import lax
from jax.experimental import pallas as pl
from jax.experimental.pallas import tpu as pltpu
from jax.experimental.pallas import tpu_sc as plsc

_TAU = 0.7
_FGT = 0.1
_MASS = 0.2
_CTX = 1.8
_NI, _NC, _H, _W = 4, 20, 512, 512
_CP = 32
_NCORE, _NSUB = 2, 16
_NW = _NCORE * _NSUB
_R = 20000
_RP = 20480
_RPW = _RP // _NW
_GRP = 16
_NGRP = _RPW // _GRP


def _ii_body(x_ref, o_ref):
    x = x_ref[0, 0]
    mx = jnp.max(x)
    fg = jnp.where(x >= _FGT * mx, x, 0.0)
    r = lax.broadcasted_iota(jnp.int32, (_H, _H), 0)
    c = lax.broadcasted_iota(jnp.int32, (_H, _H), 1)
    d = r - c
    lo = jnp.clip(d + 1, 0, 1).astype(jnp.float32)
    up = jnp.clip(1 - d, 0, 1).astype(jnp.float32)
    t = jnp.dot(lo, fg, preferred_element_type=jnp.float32)
    o_ref[0, 0] = jnp.dot(t, up, preferred_element_type=jnp.float32)


_ii_call = pl.pallas_call(
    _ii_body,
    out_shape=jax.ShapeDtypeStruct((_NI, _NC, _H, _W), jnp.float32),
    grid=(_NI, _NC),
    in_specs=[pl.BlockSpec((1, 1, _H, _W), lambda n, c: (n, c, 0, 0))],
    out_specs=pl.BlockSpec((1, 1, _H, _W), lambda n, c: (n, c, 0, 0)),
)

_YQ = 4
_QB = 16
_NBLK = _NI * (_H // (_YQ * _QB))
_TBROWS = _QB * _W
_TROWS = (_NBLK + 1) * _TBROWS
_ZROW = _NI * (_H // _YQ) * _W


def _tr_body(x_ref, o_ref):
    g = pl.program_id(0)

    @pl.when(g >= _NBLK)
    def _():
        o_ref[...] = jnp.zeros((_TBROWS, 4 * _CP), jnp.float32)

    @pl.when(g < _NBLK)
    def _():
        ri = lax.broadcasted_iota(jnp.int32, (_YQ * _NC, 4 * _CP), 0)
        ci = lax.broadcasted_iota(jnp.int32, (_YQ * _NC, 4 * _CP), 1)
        tgt = (ri % _YQ) * _CP + ri // _YQ
        sel = jnp.clip(1 - jnp.abs(tgt - ci), 0, 1).astype(jnp.bfloat16)
        dn = (((0,), (0,)), ((), ()))
        for yq in range(_QB):
            m = x_ref[0, :, yq * _YQ:(yq + 1) * _YQ, :].reshape(
                _NC * _YQ, _W)
            m_hi = m.astype(jnp.bfloat16)
            m_lo = (m - m_hi.astype(jnp.float32)).astype(jnp.bfloat16)
            o_ref[yq * _W:(yq + 1) * _W, :] = (
                lax.dot_general(m_hi, sel, dn,
                                preferred_element_type=jnp.float32)
                + lax.dot_general(m_lo, sel, dn,
                                  preferred_element_type=jnp.float32))


_tr_call = pl.pallas_call(
    _tr_body,
    out_shape=jax.ShapeDtypeStruct((_TROWS, 4 * _CP), jnp.float32),
    grid=(_NBLK + 1,),
    in_specs=[pl.BlockSpec(
        (1, _NC, _YQ * _QB, _W),
        lambda g: (jnp.minimum(g // (_H // (_YQ * _QB)), _NI - 1), 0,
                   g % (_H // (_YQ * _QB)), 0))],
    out_specs=pl.BlockSpec((_TBROWS, 4 * _CP), lambda g: (g, 0)),
)


def _bc(v16, j):
    idx = jnp.full((16, 1), j, dtype=jnp.int32)
    return lax.gather(
        v16, idx,
        lax.GatherDimensionNumbers(offset_dims=(), collapsed_slice_dims=(0,),
                                   start_index_map=(0,)),
        (1,), mode=lax.GatherScatterMode.PROMISE_IN_BOUNDS)


_mesh = plsc.VectorSubcoreMesh(core_axis_name="c", subcore_axis_name="s")


@functools.partial(
    pl.kernel,
    out_type=(jax.ShapeDtypeStruct((_NW, _RPW * _CP), jnp.float32),
              jax.ShapeDtypeStruct((_NW, 128), jnp.float32)),
    mesh=_mesh,
    scratch_types=[
        pltpu.VMEM((_RPW * 8,), jnp.int32),
        pltpu.VMEM((_RPW * 19,), jnp.float32),
        pltpu.VMEM((128,), jnp.float32),
        pltpu.VMEM((2, _GRP * 8, 4 * _CP), jnp.float32),
        pltpu.VMEM((_RPW * _CP,), jnp.float32),
        pltpu.VMEM((128,), jnp.float32),
        pltpu.SemaphoreType.DMA,
        pltpu.SemaphoreType.DMA,
    ],
)
def _csc_pass1(tbl, idxh, rph, ith, score_h, part_h,
               idx_v, rp_v, it_v, rows2_v, sc_v, pt_v, sem_a, sem_b):
    wid = lax.axis_index("s") * _NCORE + lax.axis_index("c")
    pltpu.sync_copy(idxh.at[wid], idx_v)
    pltpu.sync_copy(rph.at[wid], rp_v)
    pltpu.sync_copy(ith, it_v)
    it_rows = [(it_v[n * 32:n * 32 + 16], it_v[n * 32 + 16:n * 32 + 32])
               for n in range(4)]
    zero = jnp.zeros((16,), jnp.float32)

    def idx_at(g):
        return idx_v.at[pl.ds(g * (_GRP * 8), _GRP * 8)]

    pltpu.async_copy(tbl.at[idx_at(0)], rows2_v.at[0], sem_a)

    def grp(g, acc):
        p = g % 2

        @pl.when(jnp.logical_and(g + 1 < _NGRP, p == 0))
        def _():
            pltpu.async_copy(tbl.at[idx_at(g + 1)], rows2_v.at[1], sem_b)

        @pl.when(jnp.logical_and(g + 1 < _NGRP, p == 1))
        def _():
            pltpu.async_copy(tbl.at[idx_at(g + 1)], rows2_v.at[0], sem_a)

        @pl.when(p == 0)
        def _():
            pltpu.make_async_copy(tbl.at[idx_at(g)], rows2_v.at[0],
                                  sem_a).wait()

        @pl.when(p == 1)
        def _():
            pltpu.make_async_copy(tbl.at[idx_at(g)], rows2_v.at[1],
                                  sem_b).wait()

        rows_v = rows2_v.at[p]
        r16 = rp_v[pl.ds(g * _GRP, _GRP)]
        f16 = rp_v[pl.ds(_RPW + g * _GRP, _GRP)]
        b16 = rp_v[pl.ds(2 * _RPW + g * _GRP, _GRP)]
        q16a = [rp_v[pl.ds((3 + k) * _RPW + g * _GRP, _GRP)] for k in range(8)]
        q16b = [rp_v[pl.ds((11 + k) * _RPW + g * _GRP, _GRP)]
                for k in range(8)]
        base = g * (_GRP * _CP)
        acc = list(acc)
        for j in range(_GRP):
            rb = j * 8
            rin = _bc(r16, j)
            rfr = _bc(f16, j)
            bb = _bc(b16, j)
            fa = [_bc(q16a[k], j) for k in range(8)]
            fb = [_bc(q16b[k], j) for k in range(8)]
            for h in range(2):
                s0 = h * 16

                def pick(k, _s0=s0, _rb=rb, _fa=fa, _fb=fb):
                    r = rb + k
                    v0 = rows_v[r, _s0:_s0 + 16]
                    v1 = rows_v[r, 32 + _s0:32 + _s0 + 16]
                    v2 = rows_v[r, 64 + _s0:64 + _s0 + 16]
                    v3 = rows_v[r, 96 + _s0:96 + _s0 + 16]
                    lo = v0 + _fa[k] * (v1 - v0)
                    hi = v2 + _fa[k] * (v3 - v2)
                    return lo + _fb[k] * (hi - lo)

                cv = [pick(k) for k in range(8)]
                va, vb, vc, vd, ve, vf, vg, vh = cv
                m_in = va - vb - vc + vd
                m_ctx = ve - vf - vg + vh
                m_fr = jnp.maximum(m_ctx - m_in, 0.0)
                s = m_in * rin - m_fr * rfr
                oh = [jnp.maximum(1.0 - jnp.abs(bb - float(n)), 0.0)
                      for n in range(4)]
                itr = oh[0] * it_rows[0][h]
                for n in range(1, 4):
                    itr = itr + oh[n] * it_rows[n][h]
                gate = jnp.logical_and(m_in * itr >= _MASS, m_in >= 0.0)
                sp = jnp.where(gate, s, jnp.minimum(s, 0.0))
                sc_v[pl.ds(base + j * _CP + s0, 16)] = sp
                ab = jnp.abs(s)
                for n in range(4):
                    k = n * 2 + h
                    acc[k] = jnp.maximum(acc[k], ab * oh[n])
        return tuple(acc)

    acc = lax.fori_loop(0, _NGRP, grp, tuple(zero for _ in range(8)))
    for n in range(4):
        pt_v[n * 32:n * 32 + 16] = acc[n * 2]
        pt_v[n * 32 + 16:n * 32 + 32] = acc[n * 2 + 1]
    pltpu.sync_copy(sc_v, score_h.at[wid])
    pltpu.sync_copy(pt_v, part_h.at[wid])


@functools.partial(
    pl.kernel,
    out_type=jax.ShapeDtypeStruct((_NW, _RPW * _CP), jnp.float32),
    mesh=_mesh,
    scratch_types=[
        pltpu.VMEM((_RPW * _CP,), jnp.float32),
        pltpu.VMEM((_NW, 128), jnp.float32),
        pltpu.VMEM((_RPW * 19,), jnp.float32),
        pltpu.VMEM((128,), jnp.float32),
        pltpu.VMEM((_RPW * _CP,), jnp.float32),
    ],
)
def _csc_pass2(score_h, part_h, rph, labh, w_h, sc_v, pt_v, rp_v, lab_v, w_v):
    wid = lax.axis_index("s") * _NCORE + lax.axis_index("c")
    pltpu.sync_copy(score_h.at[wid], sc_v)
    pltpu.sync_copy(part_h, pt_v)
    pltpu.sync_copy(rph.at[wid], rp_v)
    pltpu.sync_copy(labh, lab_v)
    iam = []
    for n in range(4):
        for h in range(2):
            o = n * 32 + h * 16
            m = pt_v[0, o:o + 16]
            for t in range(1, _NW):
                m = jnp.maximum(m, pt_v[t, o:o + 16])
            iam.append(1.0 / jnp.maximum(m, 1e-6))
    lab_rows = [(lab_v[n * 32:n * 32 + 16], lab_v[n * 32 + 16:n * 32 + 32])
                for n in range(4)]

    def grp(g, carry):
        b16 = rp_v[pl.ds(2 * _RPW + g * _GRP, _GRP)]
        base = g * (_GRP * _CP)
        for j in range(_GRP):
            bb = _bc(b16, j)
            for h in range(2):
                off = base + j * _CP + h * 16
                s = sc_v[pl.ds(off, 16)]
                oh = [jnp.maximum(1.0 - jnp.abs(bb - float(n)), 0.0)
                      for n in range(4)]
                ia = oh[0] * iam[h]
                lb = oh[0] * lab_rows[0][h]
                for n in range(1, 4):
                    ia = ia + oh[n] * iam[n * 2 + h]
                    lb = lb + oh[n] * lab_rows[n][h]
                w = jnp.clip(s * ia, -1.0, 1.0)
                w = jnp.where(w >= _TAU, 1.0, w)
                w = jnp.where(lb > 0.5, w, 1.0)
                w_v[pl.ds(off, 16)] = w
        return carry

    lax.fori_loop(0, _NGRP, grp, 0)
    pltpu.sync_copy(w_v, w_h.at[wid])


def kernel(cpgs, labels, preds, rois):
    ii = _ii_call(cpgs)
    total = ii[:, :, -1, -1]
    tbl = _tr_call(ii)

    b = rois[:, 0].astype(jnp.int32)
    x1 = jnp.clip(jnp.floor(rois[:, 1]), 0, _W - 1).astype(jnp.int32)
    y1 = jnp.clip(jnp.floor(rois[:, 2]), 0, _H - 1).astype(jnp.int32)
    x2 = jnp.clip(jnp.ceil(rois[:, 3]), 0, _W - 1).astype(jnp.int32)
    y2 = jnp.clip(jnp.ceil(rois[:, 4]), 0, _H - 1).astype(jnp.int32)
    x2 = jnp.maximum(x2, x1)
    y2 = jnp.maximum(y2, y1)
    cx = (x1 + x2).astype(jnp.float32) * 0.5
    cy = (y1 + y2).astype(jnp.float32) * 0.5
    hw = (x2 - x1 + 1).astype(jnp.float32) * 0.5 * _CTX
    hh = (y2 - y1 + 1).astype(jnp.float32) * 0.5 * _CTX
    cx1 = jnp.clip(jnp.floor(cx - hw), 0, _W - 1).astype(jnp.int32)
    cy1 = jnp.clip(jnp.floor(cy - hh), 0, _H - 1).astype(jnp.int32)
    cx2 = jnp.clip(jnp.ceil(cx + hw), 0, _W - 1).astype(jnp.int32)
    cy2 = jnp.clip(jnp.ceil(cy + hh), 0, _H - 1).astype(jnp.int32)

    def rowidx(py, px):
        valid = jnp.logical_and(py > 0, px > 0)
        row = (b * (_H // _YQ) + (py - 1) // _YQ) * _W + (px - 1)
        q = jnp.where(valid, (py - 1) % _YQ, 0)
        return (jnp.where(valid, row, _ZROW),
                (q % 2).astype(jnp.float32), (q // 2).astype(jnp.float32))

    corners = [
        rowidx(y2 + 1, x2 + 1), rowidx(y1, x2 + 1),
        rowidx(y2 + 1, x1), rowidx(y1, x1),
        rowidx(cy2 + 1, cx2 + 1), rowidx(cy1, cx2 + 1),
        rowidx(cy2 + 1, cx1), rowidx(cy1, cx1),
    ]
    idx8 = jnp.stack([cr[0] for cr in corners], axis=1)
    idx8 = jnp.pad(idx8, ((0, _RP - _R), (0, 0)), constant_values=_ZROW)

    a_in = ((x2 - x1 + 1) * (y2 - y1 + 1)).astype(jnp.float32)
    a_ctx = ((cx2 - cx1 + 1) * (cy2 - cy1 + 1)).astype(jnp.float32)
    a_fr = jnp.maximum(a_ctx - a_in, 1.0)
    r_in = 1.0 / jnp.sqrt(a_in)
    r_fr = 1.0 / jnp.sqrt(a_fr)
    pad = lambda v: jnp.pad(v, (0, _RP - _R))
    slots = ([r_in, r_fr, b.astype(jnp.float32)]
             + [cr[1] for cr in corners] + [cr[2] for cr in corners])
    rp = jnp.stack([pad(v).reshape(_NW, _RPW) for v in slots],
                   axis=1).reshape(_NW, 19 * _RPW)

    inv_total = 1.0 / jnp.maximum(total, 1e-6)
    itf = jnp.pad(inv_total, ((0, 0), (0, _CP - _NC))).reshape(128)
    labf = jnp.pad(labels, ((0, 0), (0, _CP - _NC))).reshape(128)

    score_flat, part = _csc_pass1(tbl, idx8.reshape(_NW, _RPW * 8), rp, itf)
    w_flat = _csc_pass2(score_flat, part, rp, labf)
    W_out = w_flat.reshape(_RP, _CP)[:_R, :_NC]
    return (W_out, labels, jnp.zeros_like(labels))


# speedup vs baseline: 2.9137x; 1.8160x over previous
import functools

import jax
import jax.numpy as jnp
from jax import lax
from jax.experimental import pallas as pl
from jax.experimental.pallas import tpu as pltpu
from jax.experimental.pallas import tpu_sc as plsc

_TAU = 0.7
_FGT = 0.1
_MASS = 0.2
_CTX = 1.8
_NI, _NC, _H, _W = 4, 20, 512, 512
_CP = 32
_NCORE, _NSUB = 2, 16
_NW = _NCORE * _NSUB
_R = 20000
_RP = 20480
_RPW = _RP // _NW
_GRP = 16
_NGRP = _RPW // _GRP


def _ii_body(x_ref, o_ref):
    x = x_ref[0, 0]
    mx = jnp.max(x)
    fg = jnp.where(x >= _FGT * mx, x, 0.0)
    r = lax.broadcasted_iota(jnp.int32, (_H, _H), 0)
    c = lax.broadcasted_iota(jnp.int32, (_H, _H), 1)
    d = r - c
    lo = jnp.clip(d + 1, 0, 1).astype(jnp.float32)
    up = jnp.clip(1 - d, 0, 1).astype(jnp.float32)
    t = jnp.dot(lo, fg, preferred_element_type=jnp.float32)
    o_ref[0, 0] = jnp.dot(t, up, preferred_element_type=jnp.float32)


_ii_call = pl.pallas_call(
    _ii_body,
    out_shape=jax.ShapeDtypeStruct((_NI, _NC, _H, _W), jnp.float32),
    grid=(_NI, _NC),
    in_specs=[pl.BlockSpec((1, 1, _H, _W), lambda n, c: (n, c, 0, 0))],
    out_specs=pl.BlockSpec((1, 1, _H, _W), lambda n, c: (n, c, 0, 0)),
)

_YQ = 4
_QB = 16
_NBLK = _NI * (_H // (_YQ * _QB))
_TBROWS = _QB * _W
_TROWS = (_NBLK + 1) * _TBROWS
_ZROW = _NI * (_H // _YQ) * _W


def _tr_body(x_ref, o_ref):
    g = pl.program_id(0)

    @pl.when(g >= _NBLK)
    def _():
        o_ref[...] = jnp.zeros((_TBROWS, 4 * _CP), jnp.float32)

    @pl.when(g < _NBLK)
    def _():
        ri = lax.broadcasted_iota(jnp.int32, (_YQ * _NC, 4 * _CP), 0)
        ci = lax.broadcasted_iota(jnp.int32, (_YQ * _NC, 4 * _CP), 1)
        tgt = (ri % _YQ) * _CP + ri // _YQ
        sel = jnp.clip(1 - jnp.abs(tgt - ci), 0, 1).astype(jnp.bfloat16)
        dn = (((0,), (0,)), ((), ()))
        for yq in range(_QB):
            m = x_ref[0, :, yq * _YQ:(yq + 1) * _YQ, :].reshape(
                _NC * _YQ, _W)
            m_hi = m.astype(jnp.bfloat16)
            m_lo = (m - m_hi.astype(jnp.float32)).astype(jnp.bfloat16)
            o_ref[yq * _W:(yq + 1) * _W, :] = (
                lax.dot_general(m_hi, sel, dn,
                                preferred_element_type=jnp.float32)
                + lax.dot_general(m_lo, sel, dn,
                                  preferred_element_type=jnp.float32))


_tr_call = pl.pallas_call(
    _tr_body,
    out_shape=jax.ShapeDtypeStruct((_TROWS, 4 * _CP), jnp.float32),
    grid=(_NBLK + 1,),
    in_specs=[pl.BlockSpec(
        (1, _NC, _YQ * _QB, _W),
        lambda g: (jnp.minimum(g // (_H // (_YQ * _QB)), _NI - 1), 0,
                   g % (_H // (_YQ * _QB)), 0))],
    out_specs=pl.BlockSpec((_TBROWS, 4 * _CP), lambda g: (g, 0)),
)


def _bc(v16, j):
    idx = jnp.full((16, 1), j, dtype=jnp.int32)
    return lax.gather(
        v16, idx,
        lax.GatherDimensionNumbers(offset_dims=(), collapsed_slice_dims=(0,),
                                   start_index_map=(0,)),
        (1,), mode=lax.GatherScatterMode.PROMISE_IN_BOUNDS)


_mesh = plsc.VectorSubcoreMesh(core_axis_name="c", subcore_axis_name="s")


@functools.partial(
    pl.kernel,
    out_type=(jax.ShapeDtypeStruct((_NW * _RPW * _CP,), jnp.float32),
              jax.ShapeDtypeStruct((_NW * 128,), jnp.float32)),
    mesh=_mesh,
    compiler_params=pltpu.CompilerParams(use_tc_tiling_on_sc=False),
    scratch_types=[
        pltpu.VMEM((_RPW * 8,), jnp.int32),
        pltpu.VMEM((_RPW * 3,), jnp.float32),
        pltpu.VMEM((128,), jnp.float32),
        pltpu.VMEM((2, _GRP * 8, _CP), jnp.float32),
        pltpu.VMEM((_RPW * _CP,), jnp.float32),
        pltpu.VMEM((128,), jnp.float32),
        pltpu.SemaphoreType.DMA,
        pltpu.SemaphoreType.DMA,
    ],
)
def _csc_pass1(tbl, idxh, rph, ith, score_h, part_h,
               idx_v, rp_v, it_v, rows2_v, sc_v, pt_v, sem_a, sem_b):
    wid = lax.axis_index("s") * _NCORE + lax.axis_index("c")
    pltpu.sync_copy(idxh.at[pl.ds(wid * (_RPW * 8), _RPW * 8)], idx_v)
    pltpu.sync_copy(rph.at[pl.ds(wid * (_RPW * 3), _RPW * 3)], rp_v)
    pltpu.sync_copy(ith, it_v)
    it_rows = [(it_v[n * 32:n * 32 + 16], it_v[n * 32 + 16:n * 32 + 32])
               for n in range(4)]
    zero = jnp.zeros((16,), jnp.float32)

    def idx_at(g):
        return idx_v.at[pl.ds(g * (_GRP * 8), _GRP * 8)]

    pltpu.async_copy(tbl.at[idx_at(0)], rows2_v.at[0], sem_a)

    def grp(g, acc):
        p = g % 2

        @pl.when(jnp.logical_and(g + 1 < _NGRP, p == 0))
        def _():
            pltpu.async_copy(tbl.at[idx_at(g + 1)], rows2_v.at[1], sem_b)

        @pl.when(jnp.logical_and(g + 1 < _NGRP, p == 1))
        def _():
            pltpu.async_copy(tbl.at[idx_at(g + 1)], rows2_v.at[0], sem_a)

        @pl.when(p == 0)
        def _():
            pltpu.make_async_copy(tbl.at[idx_at(g)], rows2_v.at[0],
                                  sem_a).wait()

        @pl.when(p == 1)
        def _():
            pltpu.make_async_copy(tbl.at[idx_at(g)], rows2_v.at[1],
                                  sem_b).wait()

        rows_v = rows2_v.at[p]
        r16 = rp_v[pl.ds(g * _GRP, _GRP)]
        f16 = rp_v[pl.ds(_RPW + g * _GRP, _GRP)]
        b16 = rp_v[pl.ds(2 * _RPW + g * _GRP, _GRP)]
        base = g * (_GRP * _CP)
        acc = list(acc)
        for j in range(_GRP):
            rb = j * 8
            rin = _bc(r16, j)
            rfr = _bc(f16, j)
            bb = _bc(b16, j)
            for h in range(2):
                s0 = h * 16
                cv = [rows_v[rb + k, s0:s0 + 16] for k in range(8)]
                va, vb, vc, vd, ve, vf, vg, vh = cv
                m_in = va - vb - vc + vd
                m_ctx = ve - vf - vg + vh
                m_fr = jnp.maximum(m_ctx - m_in, 0.0)
                s = m_in * rin - m_fr * rfr
                oh = [jnp.maximum(1.0 - jnp.abs(bb - float(n)), 0.0)
                      for n in range(4)]
                itr = oh[0] * it_rows[0][h]
                for n in range(1, 4):
                    itr = itr + oh[n] * it_rows[n][h]
                gate = jnp.logical_and(m_in * itr >= _MASS, m_in >= 0.0)
                sp = jnp.where(gate, s, jnp.minimum(s, 0.0))
                sc_v[pl.ds(base + j * _CP + s0, 16)] = sp
                ab = jnp.abs(s)
                for n in range(4):
                    k = n * 2 + h
                    acc[k] = jnp.maximum(acc[k], ab * oh[n])
        return tuple(acc)

    acc = lax.fori_loop(0, _NGRP, grp, tuple(zero for _ in range(8)))
    for n in range(4):
        pt_v[n * 32:n * 32 + 16] = acc[n * 2]
        pt_v[n * 32 + 16:n * 32 + 32] = acc[n * 2 + 1]
    pltpu.sync_copy(sc_v, score_h.at[pl.ds(wid * (_RPW * _CP), _RPW * _CP)])
    pltpu.sync_copy(pt_v, part_h.at[pl.ds(wid * 128, 128)])


@functools.partial(
    pl.kernel,
    out_type=jax.ShapeDtypeStruct((_NW * _RPW * _CP,), jnp.float32),
    mesh=_mesh,
    compiler_params=pltpu.CompilerParams(use_tc_tiling_on_sc=False),
    scratch_types=[
        pltpu.VMEM((_RPW * _CP,), jnp.float32),
        pltpu.VMEM((_NW * 128,), jnp.float32),
        pltpu.VMEM((_RPW * 3,), jnp.float32),
        pltpu.VMEM((128,), jnp.float32),
        pltpu.VMEM((_RPW * _CP,), jnp.float32),
    ],
)
def _csc_pass2(score_h, part_h, rph, labh, w_h, sc_v, pt_v, rp_v, lab_v, w_v):
    wid = lax.axis_index("s") * _NCORE + lax.axis_index("c")
    pltpu.sync_copy(score_h.at[pl.ds(wid * (_RPW * _CP), _RPW * _CP)], sc_v)
    pltpu.sync_copy(part_h, pt_v)
    pltpu.sync_copy(rph.at[pl.ds(wid * (_RPW * 3), _RPW * 3)], rp_v)
    pltpu.sync_copy(labh, lab_v)
    iam = []
    for n in range(4):
        for h in range(2):
            o = n * 32 + h * 16
            m = pt_v[o:o + 16]
            for t in range(1, _NW):
                m = jnp.maximum(m, pt_v[t * 128 + o:t * 128 + o + 16])
            iam.append(1.0 / jnp.maximum(m, 1e-6))
    lab_rows = [(lab_v[n * 32:n * 32 + 16], lab_v[n * 32 + 16:n * 32 + 32])
                for n in range(4)]

    def grp(g, carry):
        b16 = rp_v[pl.ds(2 * _RPW + g * _GRP, _GRP)]
        base = g * (_GRP * _CP)
        for j in range(_GRP):
            bb = _bc(b16, j)
            for h in range(2):
                off = base + j * _CP + h * 16
                s = sc_v[pl.ds(off, 16)]
                oh = [jnp.maximum(1.0 - jnp.abs(bb - float(n)), 0.0)
                      for n in range(4)]
                ia = oh[0] * iam[h]
                lb = oh[0] * lab_rows[0][h]
                for n in range(1, 4):
                    ia = ia + oh[n] * iam[n * 2 + h]
                    lb = lb + oh[n] * lab_rows[n][h]
                w = jnp.clip(s * ia, -1.0, 1.0)
                w = jnp.where(w >= _TAU, 1.0, w)
                w = jnp.where(lb > 0.5, w, 1.0)
                w_v[pl.ds(off, 16)] = w
        return carry

    lax.fori_loop(0, _NGRP, grp, 0)
    pltpu.sync_copy(w_v, w_h.at[pl.ds(wid * (_RPW * _CP), _RPW * _CP)])


def kernel(cpgs, labels, preds, rois):
    ii = _ii_call(cpgs)
    total = ii[:, :, -1, -1]
    tbl = _tr_call(ii)

    b = rois[:, 0].astype(jnp.int32)
    x1 = jnp.clip(jnp.floor(rois[:, 1]), 0, _W - 1).astype(jnp.int32)
    y1 = jnp.clip(jnp.floor(rois[:, 2]), 0, _H - 1).astype(jnp.int32)
    x2 = jnp.clip(jnp.ceil(rois[:, 3]), 0, _W - 1).astype(jnp.int32)
    y2 = jnp.clip(jnp.ceil(rois[:, 4]), 0, _H - 1).astype(jnp.int32)
    x2 = jnp.maximum(x2, x1)
    y2 = jnp.maximum(y2, y1)
    cx = (x1 + x2).astype(jnp.float32) * 0.5
    cy = (y1 + y2).astype(jnp.float32) * 0.5
    hw = (x2 - x1 + 1).astype(jnp.float32) * 0.5 * _CTX
    hh = (y2 - y1 + 1).astype(jnp.float32) * 0.5 * _CTX
    cx1 = jnp.clip(jnp.floor(cx - hw), 0, _W - 1).astype(jnp.int32)
    cy1 = jnp.clip(jnp.floor(cy - hh), 0, _H - 1).astype(jnp.int32)
    cx2 = jnp.clip(jnp.ceil(cx + hw), 0, _W - 1).astype(jnp.int32)
    cy2 = jnp.clip(jnp.ceil(cy + hh), 0, _H - 1).astype(jnp.int32)

    def rowidx(py, px):
        valid = jnp.logical_and(py > 0, px > 0)
        row = (b * (_H // _YQ) + (py - 1) // _YQ) * _W + (px - 1)
        sub = row * 4 + (py - 1) % _YQ
        return jnp.where(valid, sub, 4 * _ZROW)

    idx8 = jnp.stack([
        rowidx(y2 + 1, x2 + 1), rowidx(y1, x2 + 1),
        rowidx(y2 + 1, x1), rowidx(y1, x1),
        rowidx(cy2 + 1, cx2 + 1), rowidx(cy1, cx2 + 1),
        rowidx(cy2 + 1, cx1), rowidx(cy1, cx1),
    ], axis=1)
    idx8 = jnp.pad(idx8, ((0, _RP - _R), (0, 0)),
                   constant_values=4 * _ZROW)

    a_in = ((x2 - x1 + 1) * (y2 - y1 + 1)).astype(jnp.float32)
    a_ctx = ((cx2 - cx1 + 1) * (cy2 - cy1 + 1)).astype(jnp.float32)
    a_fr = jnp.maximum(a_ctx - a_in, 1.0)
    r_in = 1.0 / jnp.sqrt(a_in)
    r_fr = 1.0 / jnp.sqrt(a_fr)
    pad = lambda v: jnp.pad(v, (0, _RP - _R))
    rp = jnp.stack([pad(r_in).reshape(_NW, _RPW),
                    pad(r_fr).reshape(_NW, _RPW),
                    pad(b.astype(jnp.float32)).reshape(_NW, _RPW)],
                   axis=1).reshape(_NW * 3 * _RPW)

    inv_total = 1.0 / jnp.maximum(total, 1e-6)
    itf = jnp.pad(inv_total, ((0, 0), (0, _CP - _NC))).reshape(128)
    labf = jnp.pad(labels, ((0, 0), (0, _CP - _NC))).reshape(128)

    tbl4 = tbl.reshape(_TROWS * 4, _CP)
    score_flat, part = _csc_pass1(tbl4, idx8.reshape(_NW * _RPW * 8),
                                  rp, itf)
    w_flat = _csc_pass2(score_flat, part, rp, labf)
    W_out = w_flat.reshape(_RP, _CP)[:_R, :_NC]
    return (W_out, labels, jnp.zeros_like(labels))
